# baseline (speedup 1.0000x reference)
import functools

import jax
import jax.numpy as jnp
import numpy as np
from jax import lax
from jax.experimental import pallas as pl

N = 8388608
ROWS = N // 128
R = 1024
GRID = ROWS // R


def _expand_matrix() -> np.ndarray:
    p = np.arange(128)[:, None]
    c = np.arange(512)[None, :]
    src = 32 * (c // 128) + (c % 128) // 4
    return (p == src).astype(np.float32)


def _onehot_body(x_ref, m_ref, o_ref):
    xb = x_ref[...]
    a = jnp.abs(xb)
    one = jnp.float32(1.0)
    zero = jnp.float32(0.0)
    idx = (jnp.where(a > 100.0, one, zero)
           + jnp.where(a > 500.0, one, zero)
           + jnp.where(a > 1000.0, one, zero))
    rep = lax.dot_general(idx.astype(jnp.bfloat16), m_ref[...],
                          (((1,), (0,)), ((), ())),
                          preferred_element_type=jnp.float32)
    c = lax.broadcasted_iota(jnp.int32, (R, 512), 1)
    cls = (c & 3).astype(jnp.float32)
    o_ref[...] = jnp.where(rep == cls, jnp.int32(1), jnp.int32(0))


@jax.jit
def kernel(x):
    x2 = x.reshape(ROWS, 128)
    m = jnp.asarray(_expand_matrix(), dtype=jnp.bfloat16)
    out = pl.pallas_call(
        _onehot_body,
        grid=(GRID,),
        in_specs=[
            pl.BlockSpec((R, 128), lambda b: (b, 0)),
            pl.BlockSpec((128, 512), lambda b: (0, 0)),
        ],
        out_specs=pl.BlockSpec((R, 512), lambda b: (b, 0)),
        out_shape=jax.ShapeDtypeStruct((ROWS, 512), jnp.int32),
    )(x2, m)
    return out.reshape(N, 4)


# speedup vs baseline: 26.8166x; 26.8166x over previous
import jax
import jax.numpy as jnp
from jax.experimental import pallas as pl

N = 8388608
ROWS = N // 128
R = 1024
GRID = ROWS // R


def _onehot_body(x_ref, o_ref):
    a = jnp.abs(x_ref[...])
    t1 = a > 100.0
    t2 = a > 500.0
    t3 = a > 1000.0
    one = jnp.int32(1)
    zero = jnp.int32(0)
    c0 = jnp.where(t1, zero, one)
    c1 = jnp.where(t1 & (~t2), one, zero)
    c2 = jnp.where(t2 & (~t3), one, zero)
    c3 = jnp.where(t3, one, zero)
    o_ref[...] = jnp.concatenate([c0, c1, c2, c3], axis=1)


@jax.jit
def kernel(x):
    x2 = x.reshape(ROWS, 128)
    out = pl.pallas_call(
        _onehot_body,
        grid=(GRID,),
        in_specs=[pl.BlockSpec((R, 128), lambda b: (b, 0))],
        out_specs=pl.BlockSpec((R, 512), lambda b: (b, 0)),
        out_shape=jax.ShapeDtypeStruct((ROWS, 512), jnp.int32),
    )(x2)
    return (out.reshape(ROWS, 4, 128)
               .transpose(0, 2, 1)
               .reshape(N, 4))


# speedup vs baseline: 29.2607x; 1.0911x over previous
import jax
import jax.numpy as jnp
from jax.experimental import pallas as pl

N = 8388608
ROWS = N // 128
R = 4096
GRID = ROWS // R


def _onehot_body(x_ref, o_ref):
    a = jnp.abs(x_ref[...])
    t1 = a > 100.0
    t2 = a > 500.0
    t3 = a > 1000.0
    one = jnp.int32(1)
    zero = jnp.int32(0)
    c0 = jnp.where(t1, zero, one)
    c1 = jnp.where(t1 & (~t2), one, zero)
    c2 = jnp.where(t2 & (~t3), one, zero)
    c3 = jnp.where(t3, one, zero)
    o_ref[...] = jnp.concatenate([c0, c1, c2, c3], axis=1)


@jax.jit
def kernel(x):
    x2 = x.reshape(ROWS, 128)
    out = pl.pallas_call(
        _onehot_body,
        grid=(GRID,),
        in_specs=[pl.BlockSpec((R, 128), lambda b: (b, 0))],
        out_specs=pl.BlockSpec((R, 512), lambda b: (b, 0)),
        out_shape=jax.ShapeDtypeStruct((ROWS, 512), jnp.int32),
    )(x2)
    return (out.reshape(ROWS, 4, 128)
               .transpose(0, 2, 1)
               .reshape(N, 4))


# speedup vs baseline: 29.4456x; 1.0063x over previous
import jax
import jax.numpy as jnp
from jax.experimental import pallas as pl

N = 8388608
ROWS = N // 128
R = 8192
GRID = ROWS // R


def _onehot_body(x_ref, o_ref):
    a = jnp.abs(x_ref[...])
    t1 = a > 100.0
    t2 = a > 500.0
    t3 = a > 1000.0
    one = jnp.int32(1)
    zero = jnp.int32(0)
    c0 = jnp.where(t1, zero, one)
    c1 = jnp.where(t1 & (~t2), one, zero)
    c2 = jnp.where(t2 & (~t3), one, zero)
    c3 = jnp.where(t3, one, zero)
    o_ref[...] = jnp.concatenate([c0, c1, c2, c3], axis=1)


@jax.jit
def kernel(x):
    x2 = x.reshape(ROWS, 128)
    out = pl.pallas_call(
        _onehot_body,
        grid=(GRID,),
        in_specs=[pl.BlockSpec((R, 128), lambda b: (b, 0))],
        out_specs=pl.BlockSpec((R, 512), lambda b: (b, 0)),
        out_shape=jax.ShapeDtypeStruct((ROWS, 512), jnp.int32),
    )(x2)
    return (out.reshape(ROWS, 4, 128)
               .transpose(0, 2, 1)
               .reshape(N, 4))


# speedup vs baseline: 30.3776x; 1.0317x over previous
import functools

import jax
import jax.numpy as jnp
from jax import lax
from jax.experimental import pallas as pl
from jax.experimental.pallas import tpu as pltpu
from jax.experimental.pallas import tpu_sc as plsc

N = 8388608
NW = 32
PER_W = N // NW
CH = 8192
NCHUNK = PER_W // CH
GROUPS = CH // 128

_mesh = plsc.VectorSubcoreMesh(core_axis_name="c", subcore_axis_name="s")


def _sc_body(x_hbm, out_hbm, xbuf, obuf, insem, outsem):
    wid = lax.axis_index("s") * 2 + lax.axis_index("c")
    xbase = wid * PER_W
    obase = wid * (PER_W * 4)

    def compute_chunk(slot):
        def group_body(g, _):
            for u in range(8):
                v = xbuf[slot, pl.ds(g * 128 + u * 16, 16)]
                a = jnp.abs(v)
                one = jnp.int32(1)
                zero = jnp.int32(0)
                s1 = jnp.where(a > 100.0, one, zero)
                s2 = jnp.where(a > 500.0, one, zero)
                s3 = jnp.where(a > 1000.0, one, zero)
                base = g * 512 + u * 16
                obuf[slot, pl.ds(base, 16)] = one - s1
                obuf[slot, pl.ds(base + 128, 16)] = s1 - s2
                obuf[slot, pl.ds(base + 256, 16)] = s2 - s3
                obuf[slot, pl.ds(base + 384, 16)] = s3
            return 0

        lax.fori_loop(0, GROUPS, group_body, 0)

    def in_copy(g, slot):
        return pltpu.make_async_copy(
            x_hbm.at[pl.ds(xbase + g * CH, CH)], xbuf.at[slot], insem.at[slot])

    def out_copy(g, slot):
        return pltpu.make_async_copy(
            obuf.at[slot], out_hbm.at[pl.ds(obase + g * CH * 4, CH * 4)],
            outsem.at[slot])

    in_copy(0, 0).start()
    for g in range(NCHUNK):
        slot = g % 2
        if g + 1 < NCHUNK:
            in_copy(g + 1, (g + 1) % 2).start()
        in_copy(g, slot).wait()
        if g >= 2:
            out_copy(g - 2, slot).wait()
        compute_chunk(slot)
        out_copy(g, slot).start()
    out_copy(NCHUNK - 2, 0).wait()
    out_copy(NCHUNK - 1, 1).wait()


@jax.jit
def kernel(x):
    sc_call = functools.partial(
        pl.kernel,
        mesh=_mesh,
        out_type=jax.ShapeDtypeStruct((N * 4,), jnp.int32),
        scratch_types=[
            pltpu.VMEM((2, CH), jnp.float32),
            pltpu.VMEM((2, CH * 4), jnp.int32),
            pltpu.SemaphoreType.DMA((2,)),
            pltpu.SemaphoreType.DMA((2,)),
        ],
    )(_sc_body)
    out = sc_call(x)
    return (out.reshape(N // 128, 4, 128)
               .transpose(0, 2, 1)
               .reshape(N, 4))


# speedup vs baseline: 46.7591x; 1.5393x over previous
import functools

import jax
import jax.numpy as jnp
from jax import lax
from jax.experimental import pallas as pl
from jax.experimental.pallas import tpu as pltpu
from jax.experimental.pallas import tpu_sc as plsc

N = 8388608
NW = 32
PER_W = N // NW
CH = 8192
NCHUNK = PER_W // CH
GROUPS = CH // 128

_mesh = plsc.VectorSubcoreMesh(core_axis_name="c", subcore_axis_name="s")


def _sc_body(x_hbm, out_hbm, xbuf, obuf, insem, outsem):
    wid = lax.axis_index("s") * 2 + lax.axis_index("c")
    xbase = wid * PER_W
    obase = wid * (PER_W * 4)

    def compute_chunk(slot):
        @plsc.parallel_loop(0, GROUPS, step=1, unroll=2)
        def group_body(g):
            for u in range(8):
                v = xbuf[slot, pl.ds(g * 128 + u * 16, 16)]
                a = jnp.abs(v)
                one = jnp.int32(1)
                zero = jnp.int32(0)
                s1 = jnp.where(a > 100.0, one, zero)
                s2 = jnp.where(a > 500.0, one, zero)
                s3 = jnp.where(a > 1000.0, one, zero)
                base = g * 512 + u * 16
                obuf[slot, pl.ds(base, 16)] = one - s1
                obuf[slot, pl.ds(base + 128, 16)] = s1 - s2
                obuf[slot, pl.ds(base + 256, 16)] = s2 - s3
                obuf[slot, pl.ds(base + 384, 16)] = s3

    def in_copy(g, slot):
        return pltpu.make_async_copy(
            x_hbm.at[pl.ds(xbase + g * CH, CH)], xbuf.at[slot], insem.at[slot])

    def out_copy(g, slot):
        return pltpu.make_async_copy(
            obuf.at[slot], out_hbm.at[pl.ds(obase + g * CH * 4, CH * 4)],
            outsem.at[slot])

    in_copy(0, 0).start()
    for g in range(NCHUNK):
        slot = g % 2
        if g + 1 < NCHUNK:
            in_copy(g + 1, (g + 1) % 2).start()
        in_copy(g, slot).wait()
        if g >= 2:
            out_copy(g - 2, slot).wait()
        compute_chunk(slot)
        out_copy(g, slot).start()
    out_copy(NCHUNK - 2, 0).wait()
    out_copy(NCHUNK - 1, 1).wait()


@jax.jit
def kernel(x):
    sc_call = functools.partial(
        pl.kernel,
        mesh=_mesh,
        out_type=jax.ShapeDtypeStruct((N * 4,), jnp.int32),
        scratch_types=[
            pltpu.VMEM((2, CH), jnp.float32),
            pltpu.VMEM((2, CH * 4), jnp.int32),
            pltpu.SemaphoreType.DMA((2,)),
            pltpu.SemaphoreType.DMA((2,)),
        ],
    )(_sc_body)
    out = sc_call(x)
    return (out.reshape(N // 128, 4, 128)
               .transpose(0, 2, 1)
               .reshape(N, 4))


# speedup vs baseline: 52.0262x; 1.1126x over previous
import functools

import jax
import jax.numpy as jnp
from jax import lax
from jax.experimental import pallas as pl
from jax.experimental.pallas import tpu as pltpu
from jax.experimental.pallas import tpu_sc as plsc

N = 8388608
NW = 32
PER_W = N // NW
CH = 8192
NCHUNK = PER_W // CH
GROUPS = CH // 128

_mesh = plsc.VectorSubcoreMesh(core_axis_name="c", subcore_axis_name="s")


def _sc_body(x_hbm, out_hbm, xbuf, obuf, insem, outsem):
    wid = lax.axis_index("s") * 2 + lax.axis_index("c")
    xbase = wid * PER_W
    obase = wid * (PER_W * 4)

    def compute_chunk(slot):
        @plsc.parallel_loop(0, GROUPS, step=1, unroll=4)
        def group_body(g):
            for u in range(8):
                v = xbuf[slot, pl.ds(g * 128 + u * 16, 16)]
                a = jnp.abs(v)
                one = jnp.int32(1)
                zero = jnp.int32(0)
                s1 = jnp.where(a > 100.0, one, zero)
                s2 = jnp.where(a > 500.0, one, zero)
                s3 = jnp.where(a > 1000.0, one, zero)
                base = g * 512 + u * 16
                obuf[slot, pl.ds(base, 16)] = one - s1
                obuf[slot, pl.ds(base + 128, 16)] = s1 - s2
                obuf[slot, pl.ds(base + 256, 16)] = s2 - s3
                obuf[slot, pl.ds(base + 384, 16)] = s3

    def in_copy(g, slot):
        return pltpu.make_async_copy(
            x_hbm.at[pl.ds(xbase + g * CH, CH)], xbuf.at[slot], insem.at[slot])

    def out_copy(g, slot):
        return pltpu.make_async_copy(
            obuf.at[slot], out_hbm.at[pl.ds(obase + g * CH * 4, CH * 4)],
            outsem.at[slot])

    in_copy(0, 0).start()
    in_copy(1, 1).start()

    def step(gg, _):
        for slot in (0, 1):
            g = 2 * gg + slot
            in_copy(g, slot).wait()

            @pl.when(gg > 0)
            def _wait_prev_out():
                out_copy(g - 2, slot).wait()

            compute_chunk(slot)
            out_copy(g, slot).start()

            @pl.when(gg < NCHUNK // 2 - 1)
            def _prefetch_in():
                in_copy(g + 2, slot).start()
        return 0

    lax.fori_loop(0, NCHUNK // 2, step, 0)
    out_copy(NCHUNK - 2, 0).wait()
    out_copy(NCHUNK - 1, 1).wait()


@jax.jit
def kernel(x):
    sc_call = functools.partial(
        pl.kernel,
        mesh=_mesh,
        out_type=jax.ShapeDtypeStruct((N * 4,), jnp.int32),
        scratch_types=[
            pltpu.VMEM((2, CH), jnp.float32),
            pltpu.VMEM((2, CH * 4), jnp.int32),
            pltpu.SemaphoreType.DMA((2,)),
            pltpu.SemaphoreType.DMA((2,)),
        ],
    )(_sc_body)
    out = sc_call(x)
    return (out.reshape(N // 128, 4, 128)
               .transpose(0, 2, 1)
               .reshape(N, 4))
